# Initial kernel scaffold; baseline (speedup 1.0000x reference)
import functools

import jax
import jax.numpy as jnp
from jax import lax
from jax.experimental import pallas as pl
from jax.experimental.pallas import tpu as pltpu
from jax.experimental.pallas import tpu_sc as plsc

N = 10000
EMB = 128
EPS = 1e-5

NC = 2
NS = 16
NW = NC * NS
CHUNK = 128
ROWS_PER_TILE = N // NS


def _sc_segsum_body(h_hbm, src_hbm, dst_hbm, zeros_hbm, out_hbm,
                    src_v, dst_v, rows_v, acc_sh, sem, *, n_chunks):
    c = lax.axis_index("c")
    s = lax.axis_index("s")
    wid = c * NS + s

    pltpu.sync_copy(zeros_hbm.at[pl.ds(s * ROWS_PER_TILE, ROWS_PER_TILE)],
                    acc_sh.at[pl.ds(s * ROWS_PER_TILE, ROWS_PER_TILE)])
    plsc.subcore_barrier()

    base0 = wid * (n_chunks * CHUNK)

    def body(i, _):
        base = base0 + i * CHUNK
        pltpu.sync_copy(src_hbm.at[pl.ds(base, CHUNK)], src_v)
        pltpu.sync_copy(dst_hbm.at[pl.ds(base, CHUNK)], dst_v)
        pltpu.async_copy(h_hbm.at[src_v], rows_v, sem).wait()
        pltpu.sync_copy(rows_v, acc_sh.at[dst_v], add=True)
        return 0

    lax.fori_loop(0, n_chunks, body, 0)
    plsc.subcore_barrier()

    pltpu.sync_copy(acc_sh.at[pl.ds(s * ROWS_PER_TILE, ROWS_PER_TILE)],
                    out_hbm.at[c].at[pl.ds(s * ROWS_PER_TILE, ROWS_PER_TILE)])


def _make_sc_segsum(n_chunks):
    mesh = plsc.VectorSubcoreMesh(core_axis_name="c", subcore_axis_name="s")
    return pl.kernel(
        functools.partial(_sc_segsum_body, n_chunks=n_chunks),
        out_type=jax.ShapeDtypeStruct((NC, N, EMB), jnp.float32),
        mesh=mesh,
        scratch_types=[
            pltpu.VMEM((CHUNK,), jnp.int32),
            pltpu.VMEM((CHUNK,), jnp.int32),
            pltpu.VMEM((CHUNK, EMB), jnp.float32),
            pltpu.VMEM_SHARED((N + 8, EMB), jnp.float32),
            pltpu.SemaphoreType.DMA,
        ],
    )


def _dense1_body(x_ref, w_ref, b_ref, o_ref):
    o_ref[...] = (jnp.dot(x_ref[...], w_ref[...],
                          preferred_element_type=jnp.float32) + b_ref[...])


def _layer_body(p_ref, h_ref, wa_ref, ba_ref, g_ref, be_ref, wb_ref, bb_ref,
                o_ref, *, final_relu):
    agg = p_ref[0] + p_ref[1] + h_ref[...]
    z = (jnp.dot(agg, wa_ref[...], preferred_element_type=jnp.float32)
         + ba_ref[...])
    mu = jnp.mean(z, axis=0, keepdims=True)
    var = jnp.mean((z - mu) ** 2, axis=0, keepdims=True)
    z = (z - mu) * lax.rsqrt(var + EPS) * g_ref[...] + be_ref[...]
    z = jnp.maximum(z, 0.0)
    out = (jnp.dot(z, wb_ref[...], preferred_element_type=jnp.float32)
           + bb_ref[...])
    if final_relu:
        out = jnp.maximum(out, 0.0)
    o_ref[...] = out


def _dense1(x, w, b):
    return pl.pallas_call(
        _dense1_body,
        out_shape=jax.ShapeDtypeStruct((N, EMB), jnp.float32),
    )(x, w, b.reshape(1, -1))


def _layer(p, h, wa, ba, g, be, wb, bb, final_relu):
    return pl.pallas_call(
        functools.partial(_layer_body, final_relu=final_relu),
        out_shape=jax.ShapeDtypeStruct((N, EMB), jnp.float32),
    )(p, h, wa, ba.reshape(1, -1), g.reshape(1, -1), be.reshape(1, -1),
      wb, bb.reshape(1, -1))


def kernel(x, edge_index, edge_attr, W1, b1, Wa0, ba0, g0, be0, Wb0, bb0,
           Wa1, ba1, g1, be1, Wb1, bb1):
    e = edge_index.shape[1]
    epw = -(-e // (NW * CHUNK)) * CHUNK
    n_chunks = epw // CHUNK
    pad = epw * NW - e

    src = edge_index[0].astype(jnp.int32)
    dst = edge_index[1].astype(jnp.int32)
    if pad:
        src = jnp.concatenate([src, jnp.zeros((pad,), jnp.int32)])
        dst = jnp.concatenate([dst, jnp.full((pad,), N, jnp.int32)])
    zeros = jnp.zeros((N, EMB), jnp.float32)

    segsum = _make_sc_segsum(n_chunks)

    h = _dense1(x, W1, b1)
    p = segsum(h, src, dst, zeros)
    h = _layer(p, h, Wa0, ba0, g0, be0, Wb0, bb0, final_relu=True)
    p = segsum(h, src, dst, zeros)
    return _layer(p, h, Wa1, ba1, g1, be1, Wb1, bb1, final_relu=False)


# speedup vs baseline: 5.0000x; 5.0000x over previous
import functools

import jax
import jax.numpy as jnp
from jax import lax
from jax.experimental import pallas as pl
from jax.experimental.pallas import tpu as pltpu
from jax.experimental.pallas import tpu_sc as plsc

N = 10000
EMB = 128
EPS = 1e-5

NC = 2
NS = 16
NW = NC * NS
CHUNK = 128
ROWS_MAIN = (N // NS) // 8 * 8
TAIL = N - NS * ROWS_MAIN


def _sc_segsum_body(h_hbm, src_hbm, dst_hbm, zeros_hbm, out_hbm,
                    src_v, dst_v, rows_v, acc_sh, sem, *, n_chunks):
    c = lax.axis_index("c")
    s = lax.axis_index("s")
    wid = c * NS + s

    pltpu.sync_copy(zeros_hbm.at[pl.ds(s * ROWS_MAIN, ROWS_MAIN)],
                    acc_sh.at[pl.ds(s * ROWS_MAIN, ROWS_MAIN)])

    @pl.when(s == 0)
    def _():
        pltpu.sync_copy(zeros_hbm.at[pl.ds(NS * ROWS_MAIN, TAIL)],
                        acc_sh.at[pl.ds(NS * ROWS_MAIN, TAIL)])

    plsc.subcore_barrier()

    base0 = wid * (n_chunks * CHUNK)

    def body(i, _):
        base = base0 + i * CHUNK
        pltpu.sync_copy(src_hbm.at[pl.ds(base, CHUNK)], src_v)
        pltpu.sync_copy(dst_hbm.at[pl.ds(base, CHUNK)], dst_v)
        pltpu.async_copy(h_hbm.at[src_v], rows_v, sem).wait()
        pltpu.sync_copy(rows_v, acc_sh.at[dst_v], add=True)
        return 0

    lax.fori_loop(0, n_chunks, body, 0)
    plsc.subcore_barrier()

    pltpu.sync_copy(acc_sh.at[pl.ds(s * ROWS_MAIN, ROWS_MAIN)],
                    out_hbm.at[c].at[pl.ds(s * ROWS_MAIN, ROWS_MAIN)])

    @pl.when(s == 0)
    def _():
        pltpu.sync_copy(acc_sh.at[pl.ds(NS * ROWS_MAIN, TAIL)],
                        out_hbm.at[c].at[pl.ds(NS * ROWS_MAIN, TAIL)])


def _make_sc_segsum(n_chunks):
    mesh = plsc.VectorSubcoreMesh(core_axis_name="c", subcore_axis_name="s")
    return pl.kernel(
        functools.partial(_sc_segsum_body, n_chunks=n_chunks),
        out_type=jax.ShapeDtypeStruct((NC, N, EMB), jnp.float32),
        mesh=mesh,
        scratch_types=[
            pltpu.VMEM((CHUNK,), jnp.int32),
            pltpu.VMEM((CHUNK,), jnp.int32),
            pltpu.VMEM((CHUNK, EMB), jnp.float32),
            pltpu.VMEM_SHARED((N + 8, EMB), jnp.float32),
            pltpu.SemaphoreType.DMA,
        ],
    )


def _dense1_body(x_ref, w_ref, b_ref, o_ref):
    o_ref[...] = (jnp.dot(x_ref[...], w_ref[...],
                          preferred_element_type=jnp.float32) + b_ref[...])


def _layer_body(p_ref, h_ref, wa_ref, ba_ref, g_ref, be_ref, wb_ref, bb_ref,
                o_ref, *, final_relu):
    agg = p_ref[0] + p_ref[1] + h_ref[...]
    z = (jnp.dot(agg, wa_ref[...], preferred_element_type=jnp.float32)
         + ba_ref[...])
    mu = jnp.mean(z, axis=0, keepdims=True)
    var = jnp.mean((z - mu) ** 2, axis=0, keepdims=True)
    z = (z - mu) * lax.rsqrt(var + EPS) * g_ref[...] + be_ref[...]
    z = jnp.maximum(z, 0.0)
    out = (jnp.dot(z, wb_ref[...], preferred_element_type=jnp.float32)
           + bb_ref[...])
    if final_relu:
        out = jnp.maximum(out, 0.0)
    o_ref[...] = out


def _dense1(x, w, b):
    return pl.pallas_call(
        _dense1_body,
        out_shape=jax.ShapeDtypeStruct((N, EMB), jnp.float32),
    )(x, w, b.reshape(1, -1))


def _layer(p, h, wa, ba, g, be, wb, bb, final_relu):
    return pl.pallas_call(
        functools.partial(_layer_body, final_relu=final_relu),
        out_shape=jax.ShapeDtypeStruct((N, EMB), jnp.float32),
    )(p, h, wa, ba.reshape(1, -1), g.reshape(1, -1), be.reshape(1, -1),
      wb, bb.reshape(1, -1))


def kernel(x, edge_index, edge_attr, W1, b1, Wa0, ba0, g0, be0, Wb0, bb0,
           Wa1, ba1, g1, be1, Wb1, bb1):
    e = edge_index.shape[1]
    epw = -(-e // (NW * CHUNK)) * CHUNK
    n_chunks = epw // CHUNK
    pad = epw * NW - e

    src = edge_index[0].astype(jnp.int32)
    dst = edge_index[1].astype(jnp.int32)
    if pad:
        src = jnp.concatenate([src, jnp.zeros((pad,), jnp.int32)])
        dst = jnp.concatenate([dst, jnp.full((pad,), N, jnp.int32)])
    zeros = jnp.zeros((N, EMB), jnp.float32)

    segsum = _make_sc_segsum(n_chunks)

    h = _dense1(x, W1, b1)
    p = segsum(h, src, dst, zeros)
    h = _layer(p, h, Wa0, ba0, g0, be0, Wb0, bb0, final_relu=True)
    p = segsum(h, src, dst, zeros)
    return _layer(p, h, Wa1, ba1, g1, be1, Wb1, bb1, final_relu=False)
